# baseline (speedup 1.0000x reference)
import jax
import jax.numpy as jnp
from jax import lax
from jax.experimental import pallas as pl
from jax.experimental.pallas import tpu as pltpu
from jax.experimental.pallas import tpu_sc as plsc

NC = 2
NS = 16
L = 16
NW = NC * NS

NCOL = 512
NROW = (16 * 1 * 512 * 512) // NCOL
NBLK = NROW // 8

TCR = 6144
TCB = 512
SC_BLK0 = TCR // 8
SC_NB = (NBLK - SC_BLK0) // NW
CB = 4
NCH = SC_NB // CB
NGRP = NCH // 2
CVEC = NCOL // L


def _sc_body(a_hbm, b_hbm, out_hbm, abuf, bbuf, acc, sa0, sa1, sb0, sb1):
    cid = lax.axis_index("c")
    sid = lax.axis_index("s")
    wid = sid * NC + cid
    blk_base = SC_BLK0 + wid * SC_NB

    sas = (sa0, sa1)
    sbs = (sb0, sb1)

    def start(chunk, slot):
        off = blk_base + chunk * CB
        pltpu.async_copy(a_hbm.at[pl.ds(off, CB)], abuf.at[slot], sas[slot])
        pltpu.async_copy(b_hbm.at[pl.ds(off, CB)], bbuf.at[slot], sbs[slot])

    def wait(slot):
        pltpu.make_async_copy(a_hbm.at[pl.ds(0, CB)], abuf.at[slot],
                              sas[slot]).wait()
        pltpu.make_async_copy(b_hbm.at[pl.ds(0, CB)], bbuf.at[slot],
                              sbs[slot]).wait()

    start(0, 0)
    start(1, 1)

    zf = jnp.zeros((L,), jnp.float32)
    zi = jnp.zeros((L,), jnp.int32)

    def group(g, carry):
        for slot in (0, 1):
            chunk = 2 * g + slot
            wait(slot)

            def rowfn(i, c, slot=slot):
                s1, c1, s2, c2 = c
                rlin = i >> 2
                ccg = i & 3
                blk = rlin >> 3
                r = rlin & 7
                col0 = ccg * (NCOL // 4)
                for k in range(CVEC // 4):
                    a = abuf[slot, blk, r, pl.ds(col0 + k * L, L)]
                    b = bbuf[slot, blk, r, pl.ds(col0 + k * L, L)]
                    d = a - b
                    sq = d * d
                    m1 = b > 0.0
                    m2 = a > 0.0
                    s1 = s1 + jnp.where(m1, sq, 0.0)
                    s2 = s2 + jnp.where(m2, sq, 0.0)
                    c1 = c1 + plsc.all_reduce_population_count(m1)
                    c2 = c2 + plsc.all_reduce_population_count(m2)
                return (s1, c1, s2, c2)

            carry = lax.fori_loop(0, CB * 8 * 4, rowfn, carry)

            nxt = chunk + 2

            @pl.when(nxt < NCH)
            def _(nxt=nxt, slot=slot):
                start(nxt, slot)
        return carry

    s1, c1, s2, c2 = lax.fori_loop(0, NGRP, group, (zf, zi, zf, zi))

    acc[0] = s1
    acc[1] = c1.astype(jnp.float32)
    acc[2] = s2
    acc[3] = c2.astype(jnp.float32)
    pltpu.sync_copy(acc, out_hbm.at[wid])


def _sc_reduce(a3, b3):
    mesh = plsc.VectorSubcoreMesh(core_axis_name="c", subcore_axis_name="s")
    return pl.kernel(
        _sc_body,
        out_type=jax.ShapeDtypeStruct((NW, 4, L), jnp.float32),
        mesh=mesh,
        compiler_params=pltpu.CompilerParams(use_tc_tiling_on_sc=True,
                                             needs_layout_passes=False),
        scratch_types=[
            pltpu.VMEM((2, CB, 8, NCOL), jnp.float32),
            pltpu.VMEM((2, CB, 8, NCOL), jnp.float32),
            pltpu.VMEM((4, L), jnp.float32),
            pltpu.SemaphoreType.DMA,
            pltpu.SemaphoreType.DMA,
            pltpu.SemaphoreType.DMA,
            pltpu.SemaphoreType.DMA,
        ],
    )(a3, b3)


def _tc_body(a_ref, b_ref, out_ref):
    step = pl.program_id(0)

    @pl.when(step == 0)
    def _():
        out_ref[...] = jnp.zeros_like(out_ref)

    a = a_ref[...]
    b = b_ref[...]
    d = a - b
    sq = d * d
    m1 = b > 0.0
    m2 = a > 0.0
    out_ref[0:1] += jnp.sum(jnp.where(m1, sq, 0.0), axis=0, keepdims=True)
    out_ref[1:2] += jnp.sum(jnp.where(m1, 1.0, 0.0), axis=0, keepdims=True)
    out_ref[2:3] += jnp.sum(jnp.where(m2, sq, 0.0), axis=0, keepdims=True)
    out_ref[3:4] += jnp.sum(jnp.where(m2, 1.0, 0.0), axis=0, keepdims=True)


def _tc_reduce(a2d, b2d):
    return pl.pallas_call(
        _tc_body,
        grid=(TCR // TCB,),
        in_specs=[
            pl.BlockSpec((TCB, NCOL), lambda i: (i, 0)),
            pl.BlockSpec((TCB, NCOL), lambda i: (i, 0)),
        ],
        out_specs=pl.BlockSpec((4, NCOL), lambda i: (0, 0)),
        out_shape=jax.ShapeDtypeStruct((4, NCOL), jnp.float32),
        compiler_params=pltpu.CompilerParams(
            dimension_semantics=("arbitrary",)),
    )(a2d, b2d)


def _finish_body(t_ref, p_ref, out_ref):
    t = t_ref[...]
    p = p_ref[...]
    rows = lax.broadcasted_iota(jnp.int32, t.shape, 0)
    comp = lax.broadcasted_iota(jnp.int32, p.shape, 1)
    s1 = jnp.sum(jnp.where(rows == 0, t, 0.0)) + \
        jnp.sum(jnp.where(comp == 0, p, 0.0))
    n1 = jnp.sum(jnp.where(rows == 1, t, 0.0)) + \
        jnp.sum(jnp.where(comp == 1, p, 0.0)) * (1.0 / L)
    s2 = jnp.sum(jnp.where(rows == 2, t, 0.0)) + \
        jnp.sum(jnp.where(comp == 2, p, 0.0))
    n2 = jnp.sum(jnp.where(rows == 3, t, 0.0)) + \
        jnp.sum(jnp.where(comp == 3, p, 0.0)) * (1.0 / L)
    l1 = s1 / n1
    l2 = s2 / n2
    loss = l1 + l2
    col = lax.broadcasted_iota(jnp.int32, (1, 128), 1)
    out_ref[...] = jnp.where(
        col == 0, loss, jnp.where(col == 1, l1,
                                  jnp.where(col == 2, l2, 0.0)))


def _finish(tc_partials, sc_partials):
    return pl.pallas_call(
        _finish_body,
        out_shape=jax.ShapeDtypeStruct((1, 128), jnp.float32),
    )(tc_partials, sc_partials)


def kernel(inputs, targets):
    a3 = inputs.reshape(NBLK, 8, NCOL)
    b3 = targets.reshape(NBLK, 8, NCOL)
    sc_partials = _sc_reduce(a3, b3)
    if TCR > 0:
        a2d = inputs.reshape(NROW, NCOL)
        b2d = targets.reshape(NROW, NCOL)
        tc_partials = _tc_reduce(a2d, b2d)
    else:
        tc_partials = jnp.zeros((4, NCOL), jnp.float32)
    res = _finish(tc_partials, sc_partials)
    return (res[0, 0], res[0, 1], res[0, 2])


# speedup vs baseline: 1.0011x; 1.0011x over previous
import jax
import jax.numpy as jnp
from jax import lax
from jax.experimental import pallas as pl
from jax.experimental.pallas import tpu as pltpu
from jax.experimental.pallas import tpu_sc as plsc

NC = 2
NS = 16
L = 16
NW = NC * NS

NCOL = 512
NROW = (16 * 1 * 512 * 512) // NCOL
NBLK = NROW // 8

TCR = 6144
TCB = 512
SC_BLK0 = TCR // 8
SC_NB = (NBLK - SC_BLK0) // NW
CB = 4
NCH = SC_NB // CB
NGRP = NCH // 2
CVEC = NCOL // L


def _sc_body(a_hbm, b_hbm, out_hbm, abuf, bbuf, acc, sa0, sa1, sb0, sb1):
    cid = lax.axis_index("c")
    sid = lax.axis_index("s")
    wid = sid * NC + cid
    blk_base = SC_BLK0 + wid * SC_NB

    sas = (sa0, sa1)
    sbs = (sb0, sb1)

    def start(chunk, slot):
        off = blk_base + chunk * CB
        pltpu.async_copy(a_hbm.at[pl.ds(off, CB)], abuf.at[slot], sas[slot])
        pltpu.async_copy(b_hbm.at[pl.ds(off, CB)], bbuf.at[slot], sbs[slot])

    def wait(slot):
        pltpu.make_async_copy(a_hbm.at[pl.ds(0, CB)], abuf.at[slot],
                              sas[slot]).wait()
        pltpu.make_async_copy(b_hbm.at[pl.ds(0, CB)], bbuf.at[slot],
                              sbs[slot]).wait()

    start(0, 0)
    start(1, 1)

    zf = jnp.zeros((L,), jnp.float32)
    zi = jnp.zeros((L,), jnp.int32)

    def group(g, carry):
        for slot in (0, 1):
            chunk = 2 * g + slot
            wait(slot)

            def rowfn(i, c, slot=slot):
                s1, c1, s2, c2 = c
                rlin = i >> 2
                ccg = i & 3
                blk = rlin >> 3
                r = rlin & 7
                col0 = ccg * (NCOL // 4)
                for k in range(CVEC // 4):
                    a = abuf[slot, blk, r, pl.ds(col0 + k * L, L)]
                    b = bbuf[slot, blk, r, pl.ds(col0 + k * L, L)]
                    d = a - b
                    sq = d * d
                    m1 = b > 0.0
                    m2 = a > 0.0
                    s1 = s1 + jnp.where(m1, sq, 0.0)
                    s2 = s2 + jnp.where(m2, sq, 0.0)
                    c1 = c1 + plsc.all_reduce_population_count(m1)
                    c2 = c2 + plsc.all_reduce_population_count(m2)
                return (s1, c1, s2, c2)

            carry = lax.fori_loop(0, CB * 8 * 4, rowfn, carry)

            nxt = chunk + 2

            @pl.when(nxt < NCH)
            def _(nxt=nxt, slot=slot):
                start(nxt, slot)
        return carry

    s1, c1, s2, c2 = lax.fori_loop(0, NGRP, group, (zf, zi, zf, zi))

    acc[0] = s1
    acc[1] = c1.astype(jnp.float32)
    acc[2] = s2
    acc[3] = c2.astype(jnp.float32)
    pltpu.sync_copy(acc, out_hbm.at[wid])


def _sc_reduce(a3, b3):
    mesh = plsc.VectorSubcoreMesh(core_axis_name="c", subcore_axis_name="s")
    return pl.kernel(
        _sc_body,
        out_type=jax.ShapeDtypeStruct((NW, 4, L), jnp.float32),
        mesh=mesh,
        compiler_params=pltpu.CompilerParams(use_tc_tiling_on_sc=True,
                                             needs_layout_passes=False),
        scratch_types=[
            pltpu.VMEM((2, CB, 8, NCOL), jnp.float32),
            pltpu.VMEM((2, CB, 8, NCOL), jnp.float32),
            pltpu.VMEM((4, L), jnp.float32),
            pltpu.SemaphoreType.DMA,
            pltpu.SemaphoreType.DMA,
            pltpu.SemaphoreType.DMA,
            pltpu.SemaphoreType.DMA,
        ],
    )(a3, b3)


def _tc_body(a_ref, b_ref, out_ref):
    step = pl.program_id(0)

    @pl.when(step == 0)
    def _():
        out_ref[...] = jnp.zeros_like(out_ref)

    a = a_ref[...]
    b = b_ref[...]
    d = a - b
    sq = d * d
    m1 = b > 0.0
    m2 = a > 0.0
    out_ref[0:1] += jnp.sum(jnp.where(m1, sq, 0.0), axis=0, keepdims=True)
    out_ref[1:2] += jnp.sum(jnp.where(m1, 1.0, 0.0), axis=0, keepdims=True)
    out_ref[2:3] += jnp.sum(jnp.where(m2, sq, 0.0), axis=0, keepdims=True)
    out_ref[3:4] += jnp.sum(jnp.where(m2, 1.0, 0.0), axis=0, keepdims=True)


def _tc_reduce(a2d, b2d):
    return pl.pallas_call(
        _tc_body,
        grid=(TCR // TCB,),
        in_specs=[
            pl.BlockSpec((TCB, NCOL), lambda i: (i, 0)),
            pl.BlockSpec((TCB, NCOL), lambda i: (i, 0)),
        ],
        out_specs=pl.BlockSpec((4, NCOL), lambda i: (0, 0)),
        out_shape=jax.ShapeDtypeStruct((4, NCOL), jnp.float32),
        compiler_params=pltpu.CompilerParams(
            dimension_semantics=("arbitrary",)),
    )(a2d, b2d)


def _finish_body(t_ref, p_ref, out_ref):
    t = t_ref[...]
    p = p_ref[...]
    rows = lax.broadcasted_iota(jnp.int32, t.shape, 0)
    comp = lax.broadcasted_iota(jnp.int32, p.shape, 1)
    s1 = jnp.sum(jnp.where(rows == 0, t, 0.0)) + \
        jnp.sum(jnp.where(comp == 0, p, 0.0))
    n1 = jnp.sum(jnp.where(rows == 1, t, 0.0)) + \
        jnp.sum(jnp.where(comp == 1, p, 0.0)) * (1.0 / L)
    s2 = jnp.sum(jnp.where(rows == 2, t, 0.0)) + \
        jnp.sum(jnp.where(comp == 2, p, 0.0))
    n2 = jnp.sum(jnp.where(rows == 3, t, 0.0)) + \
        jnp.sum(jnp.where(comp == 3, p, 0.0)) * (1.0 / L)
    l1 = s1 / n1
    l2 = s2 / n2
    loss = l1 + l2
    col = lax.broadcasted_iota(jnp.int32, (1, 128), 1)
    out_ref[...] = jnp.where(
        col == 0, loss, jnp.where(col == 1, l1,
                                  jnp.where(col == 2, l2, 0.0)))


def _finish(tc_partials, sc_partials):
    return pl.pallas_call(
        _finish_body,
        out_shape=jax.ShapeDtypeStruct((1, 128), jnp.float32),
    )(tc_partials, sc_partials)


def kernel(inputs, targets):
    a3 = inputs.reshape(NBLK, 8, NCOL)
    b3 = targets.reshape(NBLK, 8, NCOL)
    if TCR > 0:
        a2d = inputs.reshape(NROW, NCOL)
        b2d = targets.reshape(NROW, NCOL)
        tc_partials = _tc_reduce(a2d, b2d)
    else:
        tc_partials = jnp.zeros((4, NCOL), jnp.float32)
    sc_partials = _sc_reduce(a3, b3)
    res = _finish(tc_partials, sc_partials)
    return (res[0, 0], res[0, 1], res[0, 2])
